# baseline (speedup 1.0000x reference)
import functools

import jax
import jax.numpy as jnp
from jax import lax
from jax.experimental import pallas as pl
from jax.experimental.pallas import tpu as pltpu
from jax.experimental.pallas import tpu_sc as plsc


def _sc_geometry():
    try:
        info = plsc.get_sparse_core_info()
        return info.num_cores, info.num_subcores
    except Exception:
        return 2, 16


def _embed_lookup(xg, table, pe, B, S):
    V, D = table.shape
    NW, NCH, M = xg.shape
    C = S // NW
    H = C // NCH
    NP = 3
    A = 2

    mesh = plsc.VectorSubcoreMesh(core_axis_name="c", subcore_axis_name="s")

    @functools.partial(
        pl.kernel,
        mesh=mesh,
        out_type=jax.ShapeDtypeStruct((B, S, D), jnp.float32),
        scratch_types=[
            pltpu.VMEM((NCH, M), jnp.int32),
            [pltpu.VMEM((M, D), jnp.float32)] * NP,
            pltpu.VMEM((C, D), jnp.float32),
            [pltpu.SemaphoreType.DMA] * NP,
            [pltpu.SemaphoreType.DMA] * NP,
            pltpu.SemaphoreType.DMA,
        ],
    )
    def emb(xg_hbm, table_hbm, pe_hbm, out_hbm, idx_v, rows_v, pe_v, gsem,
            wsem, ssem):
        wid = lax.axis_index("s") * NC + lax.axis_index("c")
        base = wid * C
        nj = D // 16
        d1 = pltpu.async_copy(xg_hbm.at[wid], idx_v, ssem)
        d2 = pltpu.async_copy(pe_hbm.at[pl.ds(base, C)], pe_v, ssem)

        def start_gather(g):
            pltpu.async_copy(table_hbm.at[idx_v.at[g]], rows_v[g % NP],
                             gsem[g % NP])

        def drain_gather(g):
            pltpu.make_async_copy(table_hbm.at[idx_v.at[0]], rows_v[g % NP],
                                  gsem[g % NP]).wait()

        def start_writes(g):
            p = g % NP
            for b in range(B):
                pltpu.async_copy(
                    rows_v[p].at[pl.ds(b * H, H)],
                    out_hbm.at[b, pl.ds(base + g * H, H)], wsem[p])

        def drain_writes(g):
            pltpu.make_async_copy(rows_v[g % NP],
                                  out_hbm.at[0, pl.ds(base, M)],
                                  wsem[g % NP]).wait()

        d1.wait()
        d2.wait()
        start_gather(0)
        start_gather(1)
        for g in range(NCH):
            drain_gather(g)
            if g + A < NCH:
                if g + A - NP >= 0:
                    drain_writes(g + A - NP)
                start_gather(g + A)

            p = g % NP

            def add_pe_row(r, _g=g, _p=p):
                for j in range(nj):
                    v = pe_v[_g * H + r, pl.ds(j * 16, 16)]
                    for b in range(B):
                        plsc.addupdate(
                            rows_v[_p].at[b * H + r, pl.ds(j * 16, 16)], v)

            plsc.parallel_loop(0, H, 1, unroll=1)(add_pe_row)
            start_writes(g)
        for g in range(NCH - NP, NCH):
            drain_writes(g)

    return emb(xg, table, pe)


NC, NS = _sc_geometry()


def kernel(x, table, pe):
    B, S = x.shape
    NW = NC * NS
    NCH = 8
    H = S // NW // NCH
    xg = (x.astype(jnp.int32)
          .reshape(B, NW, NCH, H)
          .transpose(1, 2, 0, 3)
          .reshape(NW, NCH, B * H))
    return _embed_lookup(xg, table, pe.astype(jnp.float32), B, S)


# speedup vs baseline: 1.1230x; 1.1230x over previous
import functools

import jax
import jax.numpy as jnp
from jax import lax
from jax.experimental import pallas as pl
from jax.experimental.pallas import tpu as pltpu
from jax.experimental.pallas import tpu_sc as plsc


def _sc_geometry():
    try:
        info = plsc.get_sparse_core_info()
        return info.num_cores, info.num_subcores
    except Exception:
        return 2, 16


def _embed_lookup(xg, table, pe, B, S):
    V, D = table.shape
    NW, NCH, M = xg.shape
    C = S // NW
    H = C // NCH
    NP = 2

    mesh = plsc.VectorSubcoreMesh(core_axis_name="c", subcore_axis_name="s")

    @functools.partial(
        pl.kernel,
        mesh=mesh,
        out_type=jax.ShapeDtypeStruct((B, S, D), jnp.float32),
        scratch_types=[
            pltpu.VMEM((NCH, M), jnp.int32),
            [pltpu.VMEM((M, D), jnp.float32)] * NP,
            pltpu.VMEM((C, D), jnp.float32),
            [pltpu.SemaphoreType.DMA] * NP,
            [pltpu.SemaphoreType.DMA] * NP,
            pltpu.SemaphoreType.DMA,
        ],
    )
    def emb(xg_hbm, table_hbm, pe_hbm, out_hbm, idx_v, rows_v, pe_v, gsem,
            wsem, ssem):
        wid = lax.axis_index("s") * NC + lax.axis_index("c")
        base = wid * C
        nj = D // 16
        d1 = pltpu.async_copy(xg_hbm.at[wid], idx_v, ssem)
        d2 = pltpu.async_copy(pe_hbm.at[pl.ds(base, C)], pe_v, ssem)

        def start_gather(g, p):
            pltpu.async_copy(table_hbm.at[idx_v.at[g]], rows_v[p], gsem[p])

        def drain_gather(p):
            pltpu.make_async_copy(table_hbm.at[idx_v.at[0]], rows_v[p],
                                  gsem[p]).wait()

        def start_writes(g, p):
            for b in range(B):
                pltpu.async_copy(
                    rows_v[p].at[pl.ds(b * H, H)],
                    out_hbm.at[b, pl.ds(base + g * H, H)], wsem[p])

        def drain_writes(p):
            pltpu.make_async_copy(rows_v[p], out_hbm.at[0, pl.ds(base, M)],
                                  wsem[p]).wait()

        def process(g, p):
            gH = g * H

            def add_pe_row(r, _p=p):
                for j in range(nj):
                    v = pe_v[gH + r, pl.ds(j * 16, 16)]
                    for b in range(B):
                        plsc.addupdate(
                            rows_v[_p].at[b * H + r, pl.ds(j * 16, 16)], v)

            plsc.parallel_loop(0, H, 1, unroll=2)(add_pe_row)
            start_writes(g, p)

        d1.wait()
        d2.wait()
        start_gather(0, 0)

        def pair(g2, carry):
            g = 2 * g2
            drain_gather(0)

            @pl.when(g2 > 0)
            def _():
                drain_writes(1)
            start_gather(g + 1, 1)
            process(g, 0)

            drain_gather(1)

            @pl.when(g2 < NCH // 2 - 1)
            def _():
                drain_writes(0)
                start_gather(g + 2, 0)
            process(g + 1, 1)
            return carry

        lax.fori_loop(0, NCH // 2, pair, 0)
        drain_writes(0)
        drain_writes(1)

    return emb(xg, table, pe)


NC, NS = _sc_geometry()


def kernel(x, table, pe):
    B, S = x.shape
    NW = NC * NS
    NCH = 8
    H = S // NW // NCH
    xg = (x.astype(jnp.int32)
          .reshape(B, NW, NCH, H)
          .transpose(1, 2, 0, 3)
          .reshape(NW, NCH, B * H))
    return _embed_lookup(xg, table, pe.astype(jnp.float32), B, S)
